# baseline (speedup 1.0000x reference)
import functools

import jax
import jax.numpy as jnp
from jax import lax
from jax.experimental import pallas as pl
from jax.experimental.pallas import tpu as pltpu
from jax.experimental.pallas import tpu_sc as plsc

N = 4608
K = 512
C = 256
P = 100
PPAD = 128
M = 8192
RB = 256
NBLK = N // RB


def _cluster_body(feats_ref, protoT_ref, idx_ref, counts_ref, seg_ref):
    p2 = jnp.sum(protoT_ref[:] * protoT_ref[:], axis=0, keepdims=True)
    counts_ref[...] = jnp.zeros((1, K), jnp.float32)
    seg_ref[...] = jnp.zeros((1, K), jnp.float32)

    def blk(b, _):
        f = feats_ref[pl.ds(b * RB, RB), :]
        f2 = jnp.sum(f * f, axis=1, keepdims=True)
        dot = jnp.dot(f, protoT_ref[:], preferred_element_type=jnp.float32)
        dist = f2 + p2 - 2.0 * dot
        minv = jnp.min(dist, axis=1, keepdims=True)
        kio = lax.broadcasted_iota(jnp.int32, (RB, K), 1)
        ids = jnp.min(jnp.where(dist == minv, kio, K), axis=1, keepdims=True)
        onehot = kio == ids
        counts_ref[...] += jnp.sum(onehot.astype(jnp.float32), axis=0,
                                   keepdims=True)
        seg_ref[...] += jnp.sum(jnp.where(onehot, minv, 0.0), axis=0,
                                keepdims=True)
        return 0

    lax.fori_loop(0, NBLK, blk, 0)

    counts = counts_ref[...]
    seg = seg_ref[...]
    pv = jnp.where(counts > 0, seg / jnp.maximum(counts * float(C), 1.0), 1.0)
    mu = jnp.sum(pv, keepdims=True) / float(K)
    var_var = jnp.sum((pv - mu) ** 2, keepdims=True) / float(K)
    cr = counts + float(N) * (0.01 + var_var)
    cr = cr / jnp.sum(cr, keepdims=True)
    cnt_f = jnp.floor(cr * float(N))
    tot = jnp.sum(cnt_f, keepdims=True)
    lane = lax.broadcasted_iota(jnp.int32, (1, K), 1)
    cnt_f = jnp.where(lane == K - 1, cnt_f + (float(N) - tot), cnt_f)
    rj = lax.broadcasted_iota(jnp.int32, (K, K), 0)
    ck = lax.broadcasted_iota(jnp.int32, (K, K), 1)
    tri = (rj <= ck).astype(jnp.float32)
    csum = jnp.dot(cnt_f, tri, preferred_element_type=jnp.float32)

    def blk2(b, _):
        iv = (lax.broadcasted_iota(jnp.int32, (RB, 1), 0)
              + b * RB).astype(jnp.float32)
        cnt = jnp.sum((csum <= iv).astype(jnp.float32), axis=1, keepdims=True)
        idx_ref[pl.ds(b * RB, RB), :] = jnp.minimum(
            cnt, float(K - 1)).astype(jnp.int32)
        return 0

    lax.fori_loop(0, NBLK, blk2, 0)


def _cmpex(x, j, fj, takemin):
    up = jnp.concatenate([x[j:, :], x[:j, :]], axis=0)
    dn = jnp.concatenate([x[x.shape[0] - j:, :], x[:x.shape[0] - j, :]], axis=0)
    partner = jnp.where(fj, up, dn)
    return jnp.where(takemin, jnp.minimum(x, partner), jnp.maximum(x, partner))


def _bk(k):
    b = 2 * k
    return min(M, ((N + b - 1) // b) * b)


def _projf_body(feats_ref, theta_ref, pf_ref):
    def proj(b, _):
        rows = pl.ds(b * RB, RB)
        pf_ref[rows, :] = jnp.dot(feats_ref[rows, :], theta_ref[:],
                                  preferred_element_type=jnp.float32)
        return 0

    lax.fori_loop(0, NBLK, proj, 0)


def _swd_body(samp_ref, noise_ref, pf_ref, theta_ref, out_ref, buf_ref):
    def proj(b, _):
        rows = pl.ds(b * RB, RB)
        s = samp_ref[rows, :] + noise_ref[rows, :]
        ssq = jnp.sum(s * s, axis=1, keepdims=True)
        s = s * lax.rsqrt(ssq)
        ps = jnp.dot(s, theta_ref[:], preferred_element_type=jnp.float32)
        buf_ref[rows, 0:PPAD] = pf_ref[rows, :]
        buf_ref[rows, PPAD:2 * PPAD] = ps
        return 0

    lax.fori_loop(0, NBLK, proj, 0)

    def sentrows(b, _):
        buf_ref[pl.ds(N + b * RB, RB), :] = jnp.full(
            (RB, 2 * PPAD), jnp.inf, jnp.float32)
        return 0

    lax.fori_loop(0, (M - N) // RB, sentrows, 0)

    TR = 256
    TC2 = 128

    def local_pass(k, kmin):
        riota = lax.broadcasted_iota(jnp.int32, (TR, TC2), 0)
        subs = []
        kk = kmin
        while kk <= k:
            for j in (128, 64, 32, 16, 8, 4, 2, 1):
                if j <= min(kk, TR) // 2:
                    fj = (riota & j) == 0
                    tm = (fj == ((riota & kk) == 0)) if kk <= TR // 2 else None
                    subs.append((kk, j, fj, tm))
            kk *= 2

        def body(t, _):
            cb = (t % 2) * TC2
            base = (t // 2) * TR
            x = buf_ref[pl.ds(base, TR), pl.ds(cb, TC2)]
            for kk, j, fj, tm in subs:
                if tm is None:
                    tm = fj == ((base & kk) == 0)
                x = _cmpex(x, j, fj, tm)
            buf_ref[pl.ds(base, TR), pl.ds(cb, TC2)] = x
            return 0

        lax.fori_loop(0, (_bk(k) // TR) * 2, body, 0)

    def cross_pass(j, k):
        if 2 * k > M:
            na = (M // 2) // TR
        else:
            na = (_bk(k) // (2 * k)) * (k // TR)

        def body(t, _):
            cb = (t % 2) * TC2
            lin = (t // 2) * TR
            blk = lin // j
            off = lin % j
            a_base = blk * 2 * j + off
            b_base = a_base + j
            a = buf_ref[pl.ds(a_base, TR), pl.ds(cb, TC2)]
            b = buf_ref[pl.ds(b_base, TR), pl.ds(cb, TC2)]
            lo = jnp.minimum(a, b)
            hi = jnp.maximum(a, b)
            dirup = (a_base & k) == 0
            buf_ref[pl.ds(a_base, TR), pl.ds(cb, TC2)] = jnp.where(dirup, lo, hi)
            buf_ref[pl.ds(b_base, TR), pl.ds(cb, TC2)] = jnp.where(dirup, hi, lo)
            return 0

        lax.fori_loop(0, na * 2, body, 0)

    def cross_pass2(j1, k):
        j2 = j1 // 2
        if 2 * k > M:
            ng = (M // 4) // TR
        else:
            ng = (_bk(k) // (2 * k)) * ((k // 2) // TR)

        def body(t, _):
            cb = (t % 2) * TC2
            lin = (t // 2) * TR
            a00 = (lin // j2) * 2 * j1 + lin % j2
            cs = pl.ds(cb, TC2)
            ra = pl.ds(a00, TR)
            rb = pl.ds(a00 + j2, TR)
            rc = pl.ds(a00 + j1, TR)
            rd = pl.ds(a00 + j1 + j2, TR)
            va, vb = buf_ref[ra, cs], buf_ref[rb, cs]
            vc, vd = buf_ref[rc, cs], buf_ref[rd, cs]
            dirup = (a00 & k) == 0

            def ce(lo_cand, hi_cand):
                lo = jnp.minimum(lo_cand, hi_cand)
                hi = jnp.maximum(lo_cand, hi_cand)
                return (jnp.where(dirup, lo, hi), jnp.where(dirup, hi, lo))

            va, vc = ce(va, vc)
            vb, vd = ce(vb, vd)
            va, vb = ce(va, vb)
            vc, vd = ce(vc, vd)
            buf_ref[ra, cs] = va
            buf_ref[rb, cs] = vb
            buf_ref[rc, cs] = vc
            buf_ref[rd, cs] = vd
            return 0

        lax.fori_loop(0, ng * 2, body, 0)

    local_pass(TR, 2)
    k = 2 * TR
    while k <= M:
        js = []
        j = k // 2
        while j >= TR:
            js.append(j)
            j //= 2
        i = 0
        while i < len(js):
            if i + 1 < len(js):
                cross_pass2(js[i], k)
                i += 2
            else:
                cross_pass(js[i], k)
                i += 1
        local_pass(k, k)
        k *= 2

    def red(b, acc):
        rows = pl.ds(b * RB, RB)
        d = buf_ref[rows, PPAD:2 * PPAD] - buf_ref[rows, 0:PPAD]
        return acc + jnp.sum(d * d)

    acc = lax.fori_loop(0, N // RB, red, jnp.float32(0.0))
    out_ref[...] = jnp.zeros((1, 1), jnp.float32) + acc / float(N)


def _make_sc_gather():
    info = plsc.get_sparse_core_info()
    nw = info.num_cores * info.num_subcores
    bpw = N // nw
    mesh = plsc.VectorSubcoreMesh(core_axis_name="c", subcore_axis_name="s")

    @functools.partial(
        pl.kernel, mesh=mesh,
        out_type=jax.ShapeDtypeStruct((N, C), jnp.float32),
        scratch_types=[
            pltpu.VMEM((bpw,), jnp.int32),
            pltpu.VMEM((bpw, C), jnp.float32),
            pltpu.SemaphoreType.DMA,
        ],
    )
    def gather_k(table_hbm, idx_hbm, out_hbm, idx_v, rows_v, sem):
        wid = lax.axis_index("s") * info.num_cores + lax.axis_index("c")
        base = wid * bpw
        pltpu.sync_copy(idx_hbm.at[pl.ds(base, bpw)], idx_v)
        pltpu.async_copy(table_hbm.at[idx_v], rows_v, sem).wait()
        pltpu.sync_copy(rows_v, out_hbm.at[pl.ds(base, bpw)])

    return gather_k


def kernel(prototypes, features, rank):
    feats = features.reshape(-1, C)
    k1, k2 = jax.random.split(jax.random.key(42))
    noise = jax.random.normal(k1, (N, C), dtype=jnp.float32) / 50.0
    theta = jax.random.normal(k2, (C, P), dtype=jnp.float32)
    theta = theta / jnp.linalg.norm(theta, axis=0, keepdims=True)
    theta_pad = jnp.pad(theta, ((0, 0), (0, PPAD - P)))

    idx2d = pl.pallas_call(
        _cluster_body,
        out_shape=jax.ShapeDtypeStruct((N, 1), jnp.int32),
        scratch_shapes=[
            pltpu.VMEM((1, K), jnp.float32),
            pltpu.VMEM((1, K), jnp.float32),
        ],
    )(feats, prototypes.T)

    sampled_raw = _make_sc_gather()(prototypes, idx2d.reshape(N))

    pf = pl.pallas_call(
        _projf_body,
        out_shape=jax.ShapeDtypeStruct((N, PPAD), jnp.float32),
    )(feats, theta_pad)

    out = pl.pallas_call(
        _swd_body,
        out_shape=jax.ShapeDtypeStruct((1, 1), jnp.float32),
        scratch_shapes=[pltpu.VMEM((M, 2 * PPAD), jnp.float32)],
    )(sampled_raw, noise, pf, theta_pad)
    return out[0, 0]


# speedup vs baseline: 1.1781x; 1.1781x over previous
import functools

import jax
import jax.numpy as jnp
from jax import lax
from jax.experimental import pallas as pl
from jax.experimental.pallas import tpu as pltpu
from jax.experimental.pallas import tpu_sc as plsc

N = 4608
K = 512
C = 256
P = 100
PPAD = 128
M = 8192
RB = 256
NBLK = N // RB


def _cluster_body(feats_ref, protoT_ref, idx_ref, counts_ref, seg_ref):
    p2 = jnp.sum(protoT_ref[:] * protoT_ref[:], axis=0, keepdims=True)
    counts_ref[...] = jnp.zeros((1, K), jnp.float32)
    seg_ref[...] = jnp.zeros((1, K), jnp.float32)

    def blk(b, _):
        f = feats_ref[pl.ds(b * RB, RB), :]
        f2 = jnp.sum(f * f, axis=1, keepdims=True)
        dot = jnp.dot(f, protoT_ref[:], preferred_element_type=jnp.float32)
        dist = f2 + p2 - 2.0 * dot
        minv = jnp.min(dist, axis=1, keepdims=True)
        kio = lax.broadcasted_iota(jnp.int32, (RB, K), 1)
        ids = jnp.min(jnp.where(dist == minv, kio, K), axis=1, keepdims=True)
        onehot = kio == ids
        counts_ref[...] += jnp.sum(onehot.astype(jnp.float32), axis=0,
                                   keepdims=True)
        seg_ref[...] += jnp.sum(jnp.where(onehot, minv, 0.0), axis=0,
                                keepdims=True)
        return 0

    lax.fori_loop(0, NBLK, blk, 0)

    counts = counts_ref[...]
    seg = seg_ref[...]
    pv = jnp.where(counts > 0, seg / jnp.maximum(counts * float(C), 1.0), 1.0)
    mu = jnp.sum(pv, keepdims=True) / float(K)
    var_var = jnp.sum((pv - mu) ** 2, keepdims=True) / float(K)
    cr = counts + float(N) * (0.01 + var_var)
    cr = cr / jnp.sum(cr, keepdims=True)
    cnt_f = jnp.floor(cr * float(N))
    tot = jnp.sum(cnt_f, keepdims=True)
    lane = lax.broadcasted_iota(jnp.int32, (1, K), 1)
    cnt_f = jnp.where(lane == K - 1, cnt_f + (float(N) - tot), cnt_f)
    rj = lax.broadcasted_iota(jnp.int32, (K, K), 0)
    ck = lax.broadcasted_iota(jnp.int32, (K, K), 1)
    tri = (rj <= ck).astype(jnp.float32)
    csum = jnp.dot(cnt_f, tri, preferred_element_type=jnp.float32)

    def blk2(b, _):
        iv = (lax.broadcasted_iota(jnp.int32, (RB, 1), 0)
              + b * RB).astype(jnp.float32)
        cnt = jnp.sum((csum <= iv).astype(jnp.float32), axis=1, keepdims=True)
        idx_ref[pl.ds(b * RB, RB), :] = jnp.minimum(
            cnt, float(K - 1)).astype(jnp.int32)
        return 0

    lax.fori_loop(0, NBLK, blk2, 0)


def _cmpex(x, j, fj, takemin):
    up = jnp.concatenate([x[j:, :], x[:j, :]], axis=0)
    dn = jnp.concatenate([x[x.shape[0] - j:, :], x[:x.shape[0] - j, :]], axis=0)
    partner = jnp.where(fj, up, dn)
    return jnp.where(takemin, jnp.minimum(x, partner), jnp.maximum(x, partner))


def _cmpex_dir(x, j, fj, asc):
    up = jnp.concatenate([x[j:, :], x[:j, :]], axis=0)
    dn = jnp.concatenate([x[x.shape[0] - j:, :], x[:x.shape[0] - j, :]], axis=0)
    if asc:
        return jnp.where(fj, jnp.minimum(x, up), jnp.maximum(x, dn))
    return jnp.where(fj, jnp.maximum(x, up), jnp.minimum(x, dn))


def _bk(k):
    b = 2 * k
    return min(M, ((N + b - 1) // b) * b)


def _swd_body(samp_ref, noise_ref, feats_ref, theta_ref, out_ref, buf_ref):
    def proj(b, _):
        rows = pl.ds(b * RB, RB)
        s = samp_ref[rows, :] + noise_ref[rows, :]
        ssq = jnp.sum(s * s, axis=1, keepdims=True)
        s = s * lax.rsqrt(ssq)
        pf = jnp.dot(feats_ref[rows, :], theta_ref[:],
                     preferred_element_type=jnp.float32)
        ps = jnp.dot(s, theta_ref[:], preferred_element_type=jnp.float32)
        buf_ref[rows, 0:PPAD] = pf
        buf_ref[rows, PPAD:2 * PPAD] = ps
        return 0

    lax.fori_loop(0, NBLK, proj, 0)

    def sentrows(b, _):
        buf_ref[pl.ds(N + b * RB, RB), :] = jnp.full(
            (RB, 2 * PPAD), jnp.inf, jnp.float32)
        return 0

    lax.fori_loop(0, (M - N) // RB, sentrows, 0)

    TR = 256
    TC2 = 128

    def local_pass(k, kmin):
        riota = lax.broadcasted_iota(jnp.int32, (TR, TC2), 0)
        subs = []
        kk = kmin
        while kk <= k:
            for j in (128, 64, 32, 16, 8, 4, 2, 1):
                if j <= min(kk, TR) // 2:
                    fj = (riota & j) == 0
                    tm = (fj == ((riota & kk) == 0)) if kk <= TR // 2 else None
                    subs.append((kk, j, fj, tm))
            kk *= 2

        def body(t, _):
            cb = (t % 2) * TC2
            base = (t // 2) * TR
            x = buf_ref[pl.ds(base, TR), pl.ds(cb, TC2)]
            for kk, j, fj, tm in subs:
                if tm is None:
                    tm = fj == ((base & kk) == 0)
                x = _cmpex(x, j, fj, tm)
            buf_ref[pl.ds(base, TR), pl.ds(cb, TC2)] = x
            return 0

        lax.fori_loop(0, (_bk(k) // TR) * 2, body, 0)

    def nblocks(k):
        return max(1, _bk(k) // (2 * k))

    def local_pass_dir(k, asc):
        riota = lax.broadcasted_iota(jnp.int32, (TR, TC2), 0)
        subs = []
        for j in (128, 64, 32, 16, 8, 4, 2, 1):
            if j <= TR // 2:
                subs.append((j, (riota & j) == 0))
        tpk = k // TR
        nt = nblocks(k) * tpk
        if 2 * k > M and not asc:
            return

        def body(t, _):
            cb = (t % 2) * TC2
            tt = t // 2
            base = pl.multiple_of(
                (tt // tpk) * 2 * k + (tt % tpk) * TR + (0 if asc else k), TR)
            x = buf_ref[pl.ds(base, TR), pl.ds(cb, TC2)]
            for j, fj in subs:
                x = _cmpex_dir(x, j, fj, asc)
            buf_ref[pl.ds(base, TR), pl.ds(cb, TC2)] = x
            return 0

        lax.fori_loop(0, nt * 2, body, 0)

    def cross_pass_dir(j, k, asc):
        if 2 * k > M and not asc:
            return
        per2k = k // 2
        nt = nblocks(k) * (per2k // TR)

        def body(t, _):
            cb = (t % 2) * TC2
            lin = (t // 2) * TR
            blk = lin // per2k
            q = lin % per2k
            a_base = pl.multiple_of(
                blk * 2 * k + (q // j) * 2 * j + q % j + (0 if asc else k), TR)
            b_base = pl.multiple_of(a_base + j, TR)
            cs = pl.ds(cb, TC2)
            a = buf_ref[pl.ds(a_base, TR), cs]
            b = buf_ref[pl.ds(b_base, TR), cs]
            if asc:
                buf_ref[pl.ds(a_base, TR), cs] = jnp.minimum(a, b)
                buf_ref[pl.ds(b_base, TR), cs] = jnp.maximum(a, b)
            else:
                buf_ref[pl.ds(a_base, TR), cs] = jnp.maximum(a, b)
                buf_ref[pl.ds(b_base, TR), cs] = jnp.minimum(a, b)
            return 0

        lax.fori_loop(0, nt * 2, body, 0)

    def cross_pass2_dir(j1, k, asc):
        if 2 * k > M and not asc:
            return
        j2 = j1 // 2
        per2k = k // 4
        nt = nblocks(k) * (per2k // TR)

        def body(t, _):
            cb = (t % 2) * TC2
            lin = (t // 2) * TR
            blk = lin // per2k
            q = lin % per2k
            a00 = pl.multiple_of(
                blk * 2 * k + (q // j2) * 2 * j1 + q % j2
                + (0 if asc else k), TR)
            cs = pl.ds(cb, TC2)
            ra = pl.ds(a00, TR)
            rb = pl.ds(pl.multiple_of(a00 + j2, TR), TR)
            rc = pl.ds(pl.multiple_of(a00 + j1, TR), TR)
            rd = pl.ds(pl.multiple_of(a00 + j1 + j2, TR), TR)
            va, vb = buf_ref[ra, cs], buf_ref[rb, cs]
            vc, vd = buf_ref[rc, cs], buf_ref[rd, cs]

            def ce(u, v):
                if asc:
                    return jnp.minimum(u, v), jnp.maximum(u, v)
                return jnp.maximum(u, v), jnp.minimum(u, v)

            va, vc = ce(va, vc)
            vb, vd = ce(vb, vd)
            va, vb = ce(va, vb)
            vc, vd = ce(vc, vd)
            buf_ref[ra, cs] = va
            buf_ref[rb, cs] = vb
            buf_ref[rc, cs] = vc
            buf_ref[rd, cs] = vd
            return 0

        lax.fori_loop(0, nt * 2, body, 0)

    local_pass(TR, 2)
    k = 2 * TR
    while k <= M:
        js = []
        j = k // 2
        while j >= TR:
            js.append(j)
            j //= 2
        for asc in (True, False):
            i = 0
            while i < len(js):
                if i + 1 < len(js):
                    cross_pass2_dir(js[i], k, asc)
                    i += 2
                else:
                    cross_pass_dir(js[i], k, asc)
                    i += 1
            local_pass_dir(k, asc)
        k *= 2

    def red(b, acc):
        rows = pl.ds(b * RB, RB)
        d = buf_ref[rows, PPAD:2 * PPAD] - buf_ref[rows, 0:PPAD]
        return acc + jnp.sum(d * d)

    acc = lax.fori_loop(0, N // RB, red, jnp.float32(0.0))
    out_ref[...] = jnp.zeros((1, 1), jnp.float32) + acc / float(N)


def _make_sc_gather():
    info = plsc.get_sparse_core_info()
    nw = info.num_cores * info.num_subcores
    bpw = N // nw
    mesh = plsc.VectorSubcoreMesh(core_axis_name="c", subcore_axis_name="s")

    @functools.partial(
        pl.kernel, mesh=mesh,
        out_type=jax.ShapeDtypeStruct((N, C), jnp.float32),
        scratch_types=[
            pltpu.VMEM((bpw,), jnp.int32),
            pltpu.VMEM((bpw, C), jnp.float32),
            pltpu.SemaphoreType.DMA,
        ],
    )
    def gather_k(table_hbm, idx_hbm, out_hbm, idx_v, rows_v, sem):
        wid = lax.axis_index("s") * info.num_cores + lax.axis_index("c")
        base = wid * bpw
        pltpu.sync_copy(idx_hbm.at[pl.ds(base, bpw)], idx_v)
        pltpu.async_copy(table_hbm.at[idx_v], rows_v, sem).wait()
        pltpu.sync_copy(rows_v, out_hbm.at[pl.ds(base, bpw)])

    return gather_k


def kernel(prototypes, features, rank):
    feats = features.reshape(-1, C)
    k1, k2 = jax.random.split(jax.random.key(42))
    noise = jax.random.normal(k1, (N, C), dtype=jnp.float32) / 50.0
    theta = jax.random.normal(k2, (C, P), dtype=jnp.float32)
    theta = theta / jnp.linalg.norm(theta, axis=0, keepdims=True)
    theta_pad = jnp.pad(theta, ((0, 0), (0, PPAD - P)))

    idx2d = pl.pallas_call(
        _cluster_body,
        out_shape=jax.ShapeDtypeStruct((N, 1), jnp.int32),
        scratch_shapes=[
            pltpu.VMEM((1, K), jnp.float32),
            pltpu.VMEM((1, K), jnp.float32),
        ],
    )(feats, prototypes.T)

    sampled_raw = _make_sc_gather()(prototypes, idx2d.reshape(N))

    out = pl.pallas_call(
        _swd_body,
        out_shape=jax.ShapeDtypeStruct((1, 1), jnp.float32),
        scratch_shapes=[pltpu.VMEM((M, 2 * PPAD), jnp.float32)],
    )(sampled_raw, noise, feats, theta_pad)
    return out[0, 0]


# speedup vs baseline: 1.2251x; 1.0399x over previous
import functools

import jax
import jax.numpy as jnp
from jax import lax
from jax.experimental import pallas as pl
from jax.experimental.pallas import tpu as pltpu
from jax.experimental.pallas import tpu_sc as plsc

N = 4608
K = 512
C = 256
P = 100
PPAD = 128
M = 8192
RB = 256
NBLK = N // RB


def _cluster_body(feats_ref, protoT_ref, idx_ref, counts_ref, seg_ref):
    p2 = jnp.sum(protoT_ref[:] * protoT_ref[:], axis=0, keepdims=True)
    counts_ref[...] = jnp.zeros((1, K), jnp.float32)
    seg_ref[...] = jnp.zeros((1, K), jnp.float32)

    def blk(b, _):
        f = feats_ref[pl.ds(b * RB, RB), :]
        f2 = jnp.sum(f * f, axis=1, keepdims=True)
        dot = jnp.dot(f, protoT_ref[:], preferred_element_type=jnp.float32)
        dist = f2 + p2 - 2.0 * dot
        minv = jnp.min(dist, axis=1, keepdims=True)
        kio = lax.broadcasted_iota(jnp.int32, (RB, K), 1)
        ids = jnp.min(jnp.where(dist == minv, kio, K), axis=1, keepdims=True)
        onehot = kio == ids
        counts_ref[...] += jnp.sum(onehot.astype(jnp.float32), axis=0,
                                   keepdims=True)
        seg_ref[...] += jnp.sum(jnp.where(onehot, minv, 0.0), axis=0,
                                keepdims=True)
        return 0

    lax.fori_loop(0, NBLK, blk, 0)

    counts = counts_ref[...]
    seg = seg_ref[...]
    pv = jnp.where(counts > 0, seg / jnp.maximum(counts * float(C), 1.0), 1.0)
    mu = jnp.sum(pv, keepdims=True) / float(K)
    var_var = jnp.sum((pv - mu) ** 2, keepdims=True) / float(K)
    cr = counts + float(N) * (0.01 + var_var)
    cr = cr / jnp.sum(cr, keepdims=True)
    cnt_f = jnp.floor(cr * float(N))
    tot = jnp.sum(cnt_f, keepdims=True)
    lane = lax.broadcasted_iota(jnp.int32, (1, K), 1)
    cnt_f = jnp.where(lane == K - 1, cnt_f + (float(N) - tot), cnt_f)
    rj = lax.broadcasted_iota(jnp.int32, (K, K), 0)
    ck = lax.broadcasted_iota(jnp.int32, (K, K), 1)
    tri = (rj <= ck).astype(jnp.float32)
    csum = jnp.dot(cnt_f, tri, preferred_element_type=jnp.float32)

    def blk2(b, _):
        iv = (lax.broadcasted_iota(jnp.int32, (RB, 1), 0)
              + b * RB).astype(jnp.float32)
        cnt = jnp.sum((csum <= iv).astype(jnp.float32), axis=1, keepdims=True)
        idx_ref[pl.ds(b * RB, RB), :] = jnp.minimum(
            cnt, float(K - 1)).astype(jnp.int32)
        return 0

    lax.fori_loop(0, NBLK, blk2, 0)


def _cmpex(x, j, fj, takemin):
    up = jnp.concatenate([x[j:, :], x[:j, :]], axis=0)
    dn = jnp.concatenate([x[x.shape[0] - j:, :], x[:x.shape[0] - j, :]], axis=0)
    partner = jnp.where(fj, up, dn)
    return jnp.where(takemin, jnp.minimum(x, partner), jnp.maximum(x, partner))


def _cmpex_dir(x, j, fj, asc):
    up = jnp.concatenate([x[j:, :], x[:j, :]], axis=0)
    dn = jnp.concatenate([x[x.shape[0] - j:, :], x[:x.shape[0] - j, :]], axis=0)
    if asc:
        return jnp.where(fj, jnp.minimum(x, up), jnp.maximum(x, dn))
    return jnp.where(fj, jnp.maximum(x, up), jnp.minimum(x, dn))


def _bk(k):
    b = 2 * k
    return min(M, ((N + b - 1) // b) * b)


def _swd_body(samp_ref, noise_ref, feats_ref, theta_ref, out_ref, buf_ref):
    riota = lax.broadcasted_iota(jnp.int32, (RB, PPAD), 0)
    msubs = []
    kk = 2
    while kk <= RB:
        for j in (128, 64, 32, 16, 8, 4, 2, 1):
            if j <= min(kk, RB) // 2:
                fj = (riota & j) == 0
                tm = (fj == ((riota & kk) == 0)) if kk < RB else None
                msubs.append((j, fj, tm))
        kk *= 2

    def mega_chain(x, asc):
        for j, fj, tm in msubs:
            if tm is None:
                x = _cmpex_dir(x, j, fj, asc)
            else:
                x = _cmpex(x, j, fj, tm)
        return x

    def proj_block(b, asc):
        rows = pl.ds(b * RB, RB)
        s = samp_ref[rows, :] + noise_ref[rows, :]
        ssq = jnp.sum(s * s, axis=1, keepdims=True)
        s = s * lax.rsqrt(ssq)
        pf = jnp.dot(feats_ref[rows, :], theta_ref[:],
                     preferred_element_type=jnp.float32)
        ps = jnp.dot(s, theta_ref[:], preferred_element_type=jnp.float32)
        buf_ref[rows, 0:PPAD] = mega_chain(pf, asc)
        buf_ref[rows, PPAD:2 * PPAD] = mega_chain(ps, asc)
        return 0

    lax.fori_loop(0, NBLK // 2, lambda t, c: proj_block(2 * t, True), 0)
    lax.fori_loop(0, NBLK // 2, lambda t, c: proj_block(2 * t + 1, False), 0)

    def sentrows(b, _):
        buf_ref[pl.ds(N + b * RB, RB), :] = jnp.full(
            (RB, 2 * PPAD), jnp.inf, jnp.float32)
        return 0

    lax.fori_loop(0, (M - N) // RB, sentrows, 0)

    TR = 256
    TC2 = 128

    def nblocks(k):
        return max(1, _bk(k) // (2 * k))

    def local_pass_dir(k, asc):
        riota = lax.broadcasted_iota(jnp.int32, (TR, TC2), 0)
        subs = []
        for j in (128, 64, 32, 16, 8, 4, 2, 1):
            if j <= TR // 2:
                subs.append((j, (riota & j) == 0))
        tpk = k // TR
        nt = nblocks(k) * tpk
        if 2 * k > M and not asc:
            return

        def body(t, _):
            cb = (t % 2) * TC2
            tt = t // 2
            base = pl.multiple_of(
                (tt // tpk) * 2 * k + (tt % tpk) * TR + (0 if asc else k), TR)
            x = buf_ref[pl.ds(base, TR), pl.ds(cb, TC2)]
            for j, fj in subs:
                x = _cmpex_dir(x, j, fj, asc)
            buf_ref[pl.ds(base, TR), pl.ds(cb, TC2)] = x
            return 0

        lax.fori_loop(0, nt * 2, body, 0)

    def cross_pass_dir(j, k, asc):
        if 2 * k > M and not asc:
            return
        per2k = k // 2
        nt = nblocks(k) * (per2k // TR)

        def body(t, _):
            cb = (t % 2) * TC2
            lin = (t // 2) * TR
            blk = lin // per2k
            q = lin % per2k
            a_base = pl.multiple_of(
                blk * 2 * k + (q // j) * 2 * j + q % j + (0 if asc else k), TR)
            b_base = pl.multiple_of(a_base + j, TR)
            cs = pl.ds(cb, TC2)
            a = buf_ref[pl.ds(a_base, TR), cs]
            b = buf_ref[pl.ds(b_base, TR), cs]
            if asc:
                buf_ref[pl.ds(a_base, TR), cs] = jnp.minimum(a, b)
                buf_ref[pl.ds(b_base, TR), cs] = jnp.maximum(a, b)
            else:
                buf_ref[pl.ds(a_base, TR), cs] = jnp.maximum(a, b)
                buf_ref[pl.ds(b_base, TR), cs] = jnp.minimum(a, b)
            return 0

        lax.fori_loop(0, nt * 2, body, 0)

    def cross_pass2_dir(j1, k, asc):
        if 2 * k > M and not asc:
            return
        j2 = j1 // 2
        per2k = k // 4
        nt = nblocks(k) * (per2k // TR)

        def body(t, _):
            cb = (t % 2) * TC2
            lin = (t // 2) * TR
            blk = lin // per2k
            q = lin % per2k
            a00 = pl.multiple_of(
                blk * 2 * k + (q // j2) * 2 * j1 + q % j2
                + (0 if asc else k), TR)
            cs = pl.ds(cb, TC2)
            ra = pl.ds(a00, TR)
            rb = pl.ds(pl.multiple_of(a00 + j2, TR), TR)
            rc = pl.ds(pl.multiple_of(a00 + j1, TR), TR)
            rd = pl.ds(pl.multiple_of(a00 + j1 + j2, TR), TR)
            va, vb = buf_ref[ra, cs], buf_ref[rb, cs]
            vc, vd = buf_ref[rc, cs], buf_ref[rd, cs]

            def ce(u, v):
                if asc:
                    return jnp.minimum(u, v), jnp.maximum(u, v)
                return jnp.maximum(u, v), jnp.minimum(u, v)

            va, vc = ce(va, vc)
            vb, vd = ce(vb, vd)
            va, vb = ce(va, vb)
            vc, vd = ce(vc, vd)
            buf_ref[ra, cs] = va
            buf_ref[rb, cs] = vb
            buf_ref[rc, cs] = vc
            buf_ref[rd, cs] = vd
            return 0

        lax.fori_loop(0, nt * 2, body, 0)

    k = 2 * TR
    while k <= M:
        js = []
        j = k // 2
        while j >= TR:
            js.append(j)
            j //= 2
        for asc in (True, False):
            i = 0
            while i < len(js):
                if i + 1 < len(js):
                    cross_pass2_dir(js[i], k, asc)
                    i += 2
                else:
                    cross_pass_dir(js[i], k, asc)
                    i += 1
            if k < M:
                local_pass_dir(k, asc)
        k *= 2

    fsubs = []
    for j in (128, 64, 32, 16, 8, 4, 2, 1):
        if j <= TR // 2:
            fsubs.append((j, (riota & j) == 0))

    def fin(t, acc):
        rows = pl.ds(t * RB, RB)
        xf = buf_ref[rows, 0:PPAD]
        xs = buf_ref[rows, PPAD:2 * PPAD]
        for j, fj in fsubs:
            xf = _cmpex_dir(xf, j, fj, True)
            xs = _cmpex_dir(xs, j, fj, True)
        d = xs - xf
        return acc + jnp.sum(d * d)

    acc = lax.fori_loop(0, N // RB, fin, jnp.float32(0.0))
    out_ref[...] = jnp.zeros((1, 1), jnp.float32) + acc / float(N)


def _make_sc_gather():
    info = plsc.get_sparse_core_info()
    nw = info.num_cores * info.num_subcores
    bpw = N // nw
    mesh = plsc.VectorSubcoreMesh(core_axis_name="c", subcore_axis_name="s")

    @functools.partial(
        pl.kernel, mesh=mesh,
        out_type=jax.ShapeDtypeStruct((N, C), jnp.float32),
        scratch_types=[
            pltpu.VMEM((bpw,), jnp.int32),
            pltpu.VMEM((bpw, C), jnp.float32),
            pltpu.SemaphoreType.DMA,
        ],
    )
    def gather_k(table_hbm, idx_hbm, out_hbm, idx_v, rows_v, sem):
        wid = lax.axis_index("s") * info.num_cores + lax.axis_index("c")
        base = wid * bpw
        pltpu.sync_copy(idx_hbm.at[pl.ds(base, bpw)], idx_v)
        pltpu.async_copy(table_hbm.at[idx_v], rows_v, sem).wait()
        pltpu.sync_copy(rows_v, out_hbm.at[pl.ds(base, bpw)])

    return gather_k


def kernel(prototypes, features, rank):
    feats = features.reshape(-1, C)
    k1, k2 = jax.random.split(jax.random.key(42))
    noise = jax.random.normal(k1, (N, C), dtype=jnp.float32) / 50.0
    theta = jax.random.normal(k2, (C, P), dtype=jnp.float32)
    theta = theta / jnp.linalg.norm(theta, axis=0, keepdims=True)
    theta_pad = jnp.pad(theta, ((0, 0), (0, PPAD - P)))

    idx2d = pl.pallas_call(
        _cluster_body,
        out_shape=jax.ShapeDtypeStruct((N, 1), jnp.int32),
        scratch_shapes=[
            pltpu.VMEM((1, K), jnp.float32),
            pltpu.VMEM((1, K), jnp.float32),
        ],
    )(feats, prototypes.T)

    sampled_raw = _make_sc_gather()(prototypes, idx2d.reshape(N))

    out = pl.pallas_call(
        _swd_body,
        out_shape=jax.ShapeDtypeStruct((1, 1), jnp.float32),
        scratch_shapes=[pltpu.VMEM((M, 2 * PPAD), jnp.float32)],
    )(sampled_raw, noise, feats, theta_pad)
    return out[0, 0]


# speedup vs baseline: 1.2760x; 1.0415x over previous
import functools

import jax
import jax.numpy as jnp
from jax import lax
from jax.experimental import pallas as pl
from jax.experimental.pallas import tpu as pltpu
from jax.experimental.pallas import tpu_sc as plsc

N = 4608
K = 512
C = 256
P = 100
PPAD = 128
M = 8192
RB = 256
NBLK = N // RB
RB1 = 512
NBLK1 = N // RB1


def _cluster_body(feats_ref, protoT_ref, idx_ref, counts_ref, seg_ref):
    p2 = jnp.sum(protoT_ref[:] * protoT_ref[:], axis=0, keepdims=True)
    counts_ref[...] = jnp.zeros((1, K), jnp.float32)
    seg_ref[...] = jnp.zeros((1, K), jnp.float32)

    def blk(b, _):
        f = feats_ref[pl.ds(b * RB1, RB1), :]
        f2 = jnp.sum(f * f, axis=1, keepdims=True)
        dot = jnp.dot(f, protoT_ref[:], preferred_element_type=jnp.float32)
        dist = f2 + p2 - 2.0 * dot
        minv = jnp.min(dist, axis=1, keepdims=True)
        kio = lax.broadcasted_iota(jnp.int32, (RB1, K), 1)
        ids = jnp.min(jnp.where(dist == minv, kio, K), axis=1, keepdims=True)
        onehot = kio == ids
        counts_ref[...] += jnp.sum(onehot.astype(jnp.float32), axis=0,
                                   keepdims=True)
        seg_ref[...] += jnp.sum(jnp.where(onehot, minv, 0.0), axis=0,
                                keepdims=True)
        return 0

    lax.fori_loop(0, NBLK1, blk, 0)

    counts = counts_ref[...]
    seg = seg_ref[...]
    pv = jnp.where(counts > 0, seg / jnp.maximum(counts * float(C), 1.0), 1.0)
    mu = jnp.sum(pv, keepdims=True) / float(K)
    var_var = jnp.sum((pv - mu) ** 2, keepdims=True) / float(K)
    cr = counts + float(N) * (0.01 + var_var)
    cr = cr / jnp.sum(cr, keepdims=True)
    cnt_f = jnp.floor(cr * float(N))
    tot = jnp.sum(cnt_f, keepdims=True)
    lane = lax.broadcasted_iota(jnp.int32, (1, K), 1)
    cnt_f = jnp.where(lane == K - 1, cnt_f + (float(N) - tot), cnt_f)
    rj = lax.broadcasted_iota(jnp.int32, (K, K), 0)
    ck = lax.broadcasted_iota(jnp.int32, (K, K), 1)
    tri = (rj <= ck).astype(jnp.float32)
    csum = jnp.dot(cnt_f, tri, preferred_element_type=jnp.float32)

    def blk2(b, _):
        iv = (lax.broadcasted_iota(jnp.int32, (RB1, 1), 0)
              + b * RB1).astype(jnp.float32)
        cnt = jnp.sum((csum <= iv).astype(jnp.float32), axis=1, keepdims=True)
        idx_ref[pl.ds(b * RB1, RB1), :] = jnp.minimum(
            cnt, float(K - 1)).astype(jnp.int32)
        return 0

    lax.fori_loop(0, NBLK1, blk2, 0)


def _cmpex(x, j, fj, takemin):
    up = jnp.concatenate([x[j:, :], x[:j, :]], axis=0)
    dn = jnp.concatenate([x[x.shape[0] - j:, :], x[:x.shape[0] - j, :]], axis=0)
    partner = jnp.where(fj, up, dn)
    return jnp.where(takemin, jnp.minimum(x, partner), jnp.maximum(x, partner))


def _cmpex_dir(x, j, fj, asc):
    up = jnp.concatenate([x[j:, :], x[:j, :]], axis=0)
    dn = jnp.concatenate([x[x.shape[0] - j:, :], x[:x.shape[0] - j, :]], axis=0)
    if asc:
        return jnp.where(fj, jnp.minimum(x, up), jnp.maximum(x, dn))
    return jnp.where(fj, jnp.maximum(x, up), jnp.minimum(x, dn))


def _bk(k):
    b = 2 * k
    return min(M, ((N + b - 1) // b) * b)


def _swd_body(samp_ref, noise_ref, feats_ref, theta_ref, out_ref, buf_ref):
    riota = lax.broadcasted_iota(jnp.int32, (RB, PPAD), 0)
    msubs = []
    kk = 2
    while kk <= RB:
        for j in (128, 64, 32, 16, 8, 4, 2, 1):
            if j <= min(kk, RB) // 2:
                fj = (riota & j) == 0
                tm = (fj == ((riota & kk) == 0)) if kk < RB else None
                msubs.append((j, fj, tm))
        kk *= 2

    def mega_chain(x, asc):
        for j, fj, tm in msubs:
            if tm is None:
                x = _cmpex_dir(x, j, fj, asc)
            else:
                x = _cmpex(x, j, fj, tm)
        return x

    def proj_block(b, asc):
        rows = pl.ds(b * RB, RB)
        s = samp_ref[rows, :] + noise_ref[rows, :]
        ssq = jnp.sum(s * s, axis=1, keepdims=True)
        s = s * lax.rsqrt(ssq)
        pf = jnp.dot(feats_ref[rows, :], theta_ref[:],
                     preferred_element_type=jnp.float32)
        ps = jnp.dot(s, theta_ref[:], preferred_element_type=jnp.float32)
        buf_ref[rows, 0:PPAD] = mega_chain(pf, asc)
        buf_ref[rows, PPAD:2 * PPAD] = mega_chain(ps, asc)
        return 0

    lax.fori_loop(0, NBLK // 2, lambda t, c: proj_block(2 * t, True), 0)
    lax.fori_loop(0, NBLK // 2, lambda t, c: proj_block(2 * t + 1, False), 0)

    def sentrows(b, _):
        buf_ref[pl.ds(N + b * RB, RB), :] = jnp.full(
            (RB, 2 * PPAD), jnp.inf, jnp.float32)
        return 0

    lax.fori_loop(0, (M - N) // RB, sentrows, 0)

    TR = 256
    TC2 = 128

    def nblocks(k):
        return max(1, _bk(k) // (2 * k))

    def local_pass_dir(k, asc):
        riota = lax.broadcasted_iota(jnp.int32, (TR, TC2), 0)
        subs = []
        for j in (128, 64, 32, 16, 8, 4, 2, 1):
            if j <= TR // 2:
                subs.append((j, (riota & j) == 0))
        tpk = k // TR
        nt = nblocks(k) * tpk
        if 2 * k > M and not asc:
            return

        def body(t, _):
            cb = (t % 2) * TC2
            tt = t // 2
            base = pl.multiple_of(
                (tt // tpk) * 2 * k + (tt % tpk) * TR + (0 if asc else k), TR)
            x = buf_ref[pl.ds(base, TR), pl.ds(cb, TC2)]
            for j, fj in subs:
                x = _cmpex_dir(x, j, fj, asc)
            buf_ref[pl.ds(base, TR), pl.ds(cb, TC2)] = x
            return 0

        lax.fori_loop(0, nt * 2, body, 0)

    def cross_pass_dir(j, k, asc):
        if 2 * k > M and not asc:
            return
        per2k = k // 2
        nt = nblocks(k) * (per2k // TR)

        def body(t, _):
            cb = (t % 2) * TC2
            lin = (t // 2) * TR
            blk = lin // per2k
            q = lin % per2k
            a_base = pl.multiple_of(
                blk * 2 * k + (q // j) * 2 * j + q % j + (0 if asc else k), TR)
            b_base = pl.multiple_of(a_base + j, TR)
            cs = pl.ds(cb, TC2)
            a = buf_ref[pl.ds(a_base, TR), cs]
            b = buf_ref[pl.ds(b_base, TR), cs]
            if asc:
                buf_ref[pl.ds(a_base, TR), cs] = jnp.minimum(a, b)
                buf_ref[pl.ds(b_base, TR), cs] = jnp.maximum(a, b)
            else:
                buf_ref[pl.ds(a_base, TR), cs] = jnp.maximum(a, b)
                buf_ref[pl.ds(b_base, TR), cs] = jnp.minimum(a, b)
            return 0

        lax.fori_loop(0, nt * 2, body, 0)

    def cross_pass2_dir(j1, k, asc):
        if 2 * k > M and not asc:
            return
        j2 = j1 // 2
        per2k = k // 4
        nt = nblocks(k) * (per2k // TR)

        def body(t, _):
            cb = (t % 2) * TC2
            lin = (t // 2) * TR
            blk = lin // per2k
            q = lin % per2k
            a00 = pl.multiple_of(
                blk * 2 * k + (q // j2) * 2 * j1 + q % j2
                + (0 if asc else k), TR)
            cs = pl.ds(cb, TC2)
            ra = pl.ds(a00, TR)
            rb = pl.ds(pl.multiple_of(a00 + j2, TR), TR)
            rc = pl.ds(pl.multiple_of(a00 + j1, TR), TR)
            rd = pl.ds(pl.multiple_of(a00 + j1 + j2, TR), TR)
            va, vb = buf_ref[ra, cs], buf_ref[rb, cs]
            vc, vd = buf_ref[rc, cs], buf_ref[rd, cs]

            def ce(u, v):
                if asc:
                    return jnp.minimum(u, v), jnp.maximum(u, v)
                return jnp.maximum(u, v), jnp.minimum(u, v)

            va, vc = ce(va, vc)
            vb, vd = ce(vb, vd)
            va, vb = ce(va, vb)
            vc, vd = ce(vc, vd)
            buf_ref[ra, cs] = va
            buf_ref[rb, cs] = vb
            buf_ref[rc, cs] = vc
            buf_ref[rd, cs] = vd
            return 0

        lax.fori_loop(0, nt * 2, body, 0)

    def cross_pass3_dir(j1, k, asc):
        if 2 * k > M and not asc:
            return
        j2 = j1 // 2
        j3 = j1 // 4
        per2k = k // 8
        nt = nblocks(k) * (per2k // TR)

        def body(t, _):
            cb = (t % 2) * TC2
            lin = (t // 2) * TR
            blk = lin // per2k
            q = lin % per2k
            a0 = pl.multiple_of(
                blk * 2 * k + (q // j3) * 2 * j1 + q % j3
                + (0 if asc else k), TR)
            cs = pl.ds(cb, TC2)
            offs = [0, j3, j2, j2 + j3, j1, j1 + j3, j1 + j2, j1 + j2 + j3]
            rs = [pl.ds(pl.multiple_of(a0 + o, TR), TR) for o in offs]
            v = [buf_ref[r, cs] for r in rs]

            def ce(u, w):
                if asc:
                    return jnp.minimum(u, w), jnp.maximum(u, w)
                return jnp.maximum(u, w), jnp.minimum(u, w)

            for lo, hi in ((0, 4), (1, 5), (2, 6), (3, 7)):
                v[lo], v[hi] = ce(v[lo], v[hi])
            for lo, hi in ((0, 2), (1, 3), (4, 6), (5, 7)):
                v[lo], v[hi] = ce(v[lo], v[hi])
            for lo, hi in ((0, 1), (2, 3), (4, 5), (6, 7)):
                v[lo], v[hi] = ce(v[lo], v[hi])
            for r, x in zip(rs, v):
                buf_ref[r, cs] = x
            return 0

        lax.fori_loop(0, nt * 2, body, 0)

    k = 2 * TR
    while k <= M:
        js = []
        j = k // 2
        while j >= TR:
            js.append(j)
            j //= 2
        for asc in (True, False):
            i = 0
            while i < len(js):
                r = len(js) - i
                if r >= 3 and r != 4:
                    cross_pass3_dir(js[i], k, asc)
                    i += 3
                elif r >= 2:
                    cross_pass2_dir(js[i], k, asc)
                    i += 2
                else:
                    cross_pass_dir(js[i], k, asc)
                    i += 1
            if k < M:
                local_pass_dir(k, asc)
        k *= 2

    fsubs = []
    for j in (128, 64, 32, 16, 8, 4, 2, 1):
        if j <= TR // 2:
            fsubs.append((j, (riota & j) == 0))

    def fin(t, acc):
        rows = pl.ds(t * RB, RB)
        xf = buf_ref[rows, 0:PPAD]
        xs = buf_ref[rows, PPAD:2 * PPAD]
        for j, fj in fsubs:
            xf = _cmpex_dir(xf, j, fj, True)
            xs = _cmpex_dir(xs, j, fj, True)
        d = xs - xf
        return acc + jnp.sum(d * d)

    acc = lax.fori_loop(0, N // RB, fin, jnp.float32(0.0))
    out_ref[...] = jnp.zeros((1, 1), jnp.float32) + acc / float(N)


def _make_sc_gather():
    info = plsc.get_sparse_core_info()
    nw = info.num_cores * info.num_subcores
    bpw = N // nw
    mesh = plsc.VectorSubcoreMesh(core_axis_name="c", subcore_axis_name="s")

    @functools.partial(
        pl.kernel, mesh=mesh,
        out_type=jax.ShapeDtypeStruct((N, C), jnp.float32),
        scratch_types=[
            pltpu.VMEM((bpw,), jnp.int32),
            pltpu.VMEM((bpw, C), jnp.float32),
            pltpu.SemaphoreType.DMA,
        ],
    )
    def gather_k(table_hbm, idx_hbm, out_hbm, idx_v, rows_v, sem):
        wid = lax.axis_index("s") * info.num_cores + lax.axis_index("c")
        base = wid * bpw
        pltpu.sync_copy(idx_hbm.at[pl.ds(base, bpw)], idx_v)
        pltpu.async_copy(table_hbm.at[idx_v], rows_v, sem).wait()
        pltpu.sync_copy(rows_v, out_hbm.at[pl.ds(base, bpw)])

    return gather_k


def kernel(prototypes, features, rank):
    feats = features.reshape(-1, C)
    k1, k2 = jax.random.split(jax.random.key(42))
    noise = jax.random.normal(k1, (N, C), dtype=jnp.float32) / 50.0
    theta = jax.random.normal(k2, (C, P), dtype=jnp.float32)
    theta = theta / jnp.linalg.norm(theta, axis=0, keepdims=True)
    theta_pad = jnp.pad(theta, ((0, 0), (0, PPAD - P)))

    idx2d = pl.pallas_call(
        _cluster_body,
        out_shape=jax.ShapeDtypeStruct((N, 1), jnp.int32),
        scratch_shapes=[
            pltpu.VMEM((1, K), jnp.float32),
            pltpu.VMEM((1, K), jnp.float32),
        ],
    )(feats, prototypes.T)

    sampled_raw = _make_sc_gather()(prototypes, idx2d.reshape(N))

    out = pl.pallas_call(
        _swd_body,
        out_shape=jax.ShapeDtypeStruct((1, 1), jnp.float32),
        scratch_shapes=[pltpu.VMEM((M, 2 * PPAD), jnp.float32)],
    )(sampled_raw, noise, feats, theta_pad)
    return out[0, 0]
